# baseline (speedup 1.0000x reference)
import jax
import jax.numpy as jnp
from jax import lax
from jax.experimental import pallas as pl
from jax.experimental.pallas import tpu as pltpu
from jax.experimental.pallas import tpu_sc as plsc

_B, _M, _D = 256, 50, 60
_H = _D // 2
_K = 6
_THR = 0.2
_L = 16
_NC, _NS = 2, 16
_NW = _NC * _NS
_RW = _B // _NW
_TV = _RW * _M * _D
_SEL = _RW * _K * _D
_NEG = float("-inf")


def _sc_body(traj_hbm, score_hbm, sel_hbm, cnt_hbm,
             traj_v, score_v, sel_v, cnt_v, kept_v, acc_ref, sem):
    wid = lax.axis_index("s") * _NC + lax.axis_index("c")
    cp = pltpu.async_copy(traj_hbm.at[pl.ds(wid * _TV, _TV)],
                          traj_v.at[pl.ds(0, _TV)], sem)
    pltpu.sync_copy(score_hbm.at[pl.ds(wid * (_RW * _M), _RW * _M)],
                    score_v.at[pl.ds(0, _RW * _M)])
    cp.wait()

    iota = lax.iota(jnp.int32, _L)
    iota2 = iota * 2
    pad_hi = iota >= (_H - _L)
    full = jnp.int32(2 * _L)
    zero = jnp.int32(0)

    def argmax_mark(s0, s1, s2, s3):
        m = jnp.maximum(jnp.maximum(s0, s1), jnp.maximum(s2, s3))
        mx = jnp.max(m)
        e0 = s0 == mx
        e1 = s1 == mx
        e2 = s2 == mx
        e3 = s3 == mx
        f0 = plsc.all_reduce_ffs(e0)
        f1 = plsc.all_reduce_ffs(e1)
        f2 = plsc.all_reduce_ffs(e2)
        f3 = plsc.all_reduce_ffs(e3)
        p0 = plsc.all_reduce_population_count(e0)
        p1 = plsc.all_reduce_population_count(e1)
        p2 = plsc.all_reduce_population_count(e2)
        c = jnp.where(p0 > zero, f0,
                      jnp.where(p1 > zero, f1 + 16,
                                jnp.where(p2 > zero, f2 + 32, f3 + 48)))
        s0 = jnp.where(iota == c, _NEG, s0)
        s1 = jnp.where(iota + 16 == c, _NEG, s1)
        s2 = jnp.where(iota + 32 == c, _NEG, s2)
        s3 = jnp.where(iota + 48 == c, _NEG, s3)
        return c[0], s0, s1, s2, s3

    def deinterleave(cb):
        xa0 = plsc.load_gather(traj_v, [cb + iota2])
        xa1 = plsc.load_gather(traj_v, [cb + 32 + iota2])
        ya0 = plsc.load_gather(traj_v, [cb + 1 + iota2])
        ya1 = plsc.load_gather(traj_v, [cb + 33 + iota2])
        return xa0, xa1, ya0, ya1

    def row_body(r, cnts):
        par = (r % 2) * (_K + 1) * 64
        para = (r % 2) * (_K + 2)
        def keep_store(slot, xa0, xa1, ya0, ya1):
            ko = par + slot * 64
            kept_v[pl.ds(ko, _L)] = xa0
            kept_v[pl.ds(ko + 16, _L)] = xa1
            kept_v[pl.ds(ko + 32, _L)] = ya0
            kept_v[pl.ds(ko + 48, _L)] = ya1
        sb = r * _M
        s0 = score_v[pl.ds(sb, _L)]
        s1 = score_v[pl.ds(sb + _L, _L)]
        s2 = score_v[pl.ds(sb + 2 * _L, _L)]
        s3 = score_v[pl.ds(sb + 3 * _L, _L)]
        s3 = jnp.where(iota < (_M - 3 * _L), s3, _NEG)

        tb_row = r * (_M * _D)
        for j in range(_K + 2):
            acc_ref[para + j] = jnp.int32(0)
        c0, s0, s1, s2, s3 = argmax_mark(s0, s1, s2, s3)
        cb0 = tb_row + c0 * _D
        acc_ref[para] = cb0
        keep_store(0, *deinterleave(cb0))

        def cond(st):
            return jnp.logical_and(st[0] < _M, st[1] < _K)

        def body(st):
            visited, cnt, s0, s1, s2, s3 = st
            c, s0, s1, s2, s3 = argmax_mark(s0, s1, s2, s3)
            cb = tb_row + c * _D
            xa0, xa1, ya0, ya1 = deinterleave(cb)
            sup = jnp.full((_L,), False)
            for j in range(_K):
                ko = par + j * 64
                xb0 = kept_v[pl.ds(ko, _L)]
                xb1 = kept_v[pl.ds(ko + 16, _L)]
                yb0 = kept_v[pl.ds(ko + 32, _L)]
                yb1 = kept_v[pl.ds(ko + 48, _L)]
                dx0 = xa0 - xb0
                dy0 = ya0 - yb0
                dx1 = xa1 - xb1
                dy1 = ya1 - yb1
                d0 = dx0 * dx0 + dy0 * dy0
                d1 = dx1 * dx1 + dy1 * dy1
                near0 = d0 < _THR
                near1 = jnp.logical_or(d1 < _THR, pad_hi)
                pc = (plsc.all_reduce_population_count(near0)
                      + plsc.all_reduce_population_count(near1))
                close = pc == full
                sup = jnp.logical_or(sup, jnp.logical_and(close, j < cnt))
            take = plsc.all_reduce_population_count(sup)[0] == zero
            slot = jnp.where(take, cnt, jnp.int32(_K))
            keep_store(slot, xa0, xa1, ya0, ya1)
            acc_ref[para + jnp.where(take, cnt, jnp.int32(_K + 1))] = cb
            cnt = cnt + take.astype(jnp.int32)
            return (visited + 1, cnt, s0, s1, s2, s3)

        st = lax.while_loop(cond, body,
                            (jnp.int32(1), jnp.int32(1), s0, s1, s2, s3))
        cntf = st[1]

        ob_row = r * (_K * _D)
        for j in range(_K):
            bj = acc_ref[para + j]
            valid = j < cntf
            for p in (0, 16, 32, 44):
                v = traj_v[pl.ds(bj + p, _L)]
                v = jnp.where(valid, v, 0.0)
                sel_v[pl.ds(ob_row + j * _D + p, _L)] = v
        return jnp.where(iota == r, cntf, cnts)

    cnts = lax.fori_loop(0, _RW, row_body, jnp.zeros((_L,), jnp.int32),
                         unroll=2)
    cnt_v[...] = cnts
    pltpu.sync_copy(sel_v, sel_hbm.at[pl.ds(wid * _SEL, _SEL)])
    pltpu.sync_copy(cnt_v.at[pl.ds(0, _RW)], cnt_hbm.at[pl.ds(wid * _RW, _RW)])


def _sc_select(traj_flat, score_flat):
    mesh = plsc.VectorSubcoreMesh(core_axis_name="c", subcore_axis_name="s",
                                  num_cores=_NC, num_subcores=_NS)
    f = pl.kernel(
        _sc_body,
        out_type=(
            jax.ShapeDtypeStruct((_B * _K * _D,), jnp.float32),
            jax.ShapeDtypeStruct((_B,), jnp.int32),
        ),
        mesh=mesh,
        scratch_types=[
            pltpu.VMEM((_TV + 64,), jnp.float32),
            pltpu.VMEM((_RW * _M + 16,), jnp.float32),
            pltpu.VMEM((_SEL,), jnp.float32),
            pltpu.VMEM((_L,), jnp.int32),
            pltpu.VMEM((2 * (_K + 1) * 64,), jnp.float32),
            pltpu.SMEM((2 * (_K + 2),), jnp.int32),
            pltpu.SemaphoreType.DMA,
        ],
        compiler_params=pltpu.CompilerParams(needs_layout_passes=False),
    )
    return f(traj_flat, score_flat)


def _tc_mask_body(sel_ref, cnt_ref, out_ref):
    sel = sel_ref[...]
    cnt = cnt_ref[...]
    bi = lax.broadcasted_iota(jnp.int32, (_B, _B), 0)
    bj = lax.broadcasted_iota(jnp.int32, (_B, _B), 1)
    m = jnp.where(bj > bi, jnp.broadcast_to(cnt, (_B, _B)), jnp.int32(_K))
    suffix_after = jnp.min(m, axis=1)
    limit = jnp.minimum(cnt[0], suffix_after)
    col = lax.broadcasted_iota(jnp.int32, (_B, _K * _D), 1) // _D
    mask = col < limit[:, None]
    out_ref[...] = jnp.where(mask, sel, 0.0)


_tc_mask = pl.pallas_call(
    _tc_mask_body,
    out_shape=jax.ShapeDtypeStruct((_B, _K * _D), jnp.float32),
)


def kernel(traj_in, score):
    sel_flat, cnt = _sc_select(traj_in.reshape(-1), score.reshape(-1))
    out = _tc_mask(sel_flat.reshape(_B, _K * _D), cnt.reshape(1, _B))
    return out.reshape(_B, _K, _D)


# speedup vs baseline: 1.0032x; 1.0032x over previous
import jax
import jax.numpy as jnp
from jax import lax
from jax.experimental import pallas as pl
from jax.experimental.pallas import tpu as pltpu
from jax.experimental.pallas import tpu_sc as plsc

_B, _M, _D = 256, 50, 60
_H = _D // 2
_K = 6
_THR = 0.2
_L = 16
_NC, _NS = 2, 16
_NW = _NC * _NS
_RW = _B // _NW
_TV = _RW * _M * _D
_SEL = _RW * _K * _D
_NEG = float("-inf")


def _sc_body(traj_hbm, score_hbm, sel_hbm, cnt_hbm,
             traj_v, score_v, sel_v, cnt_v, kept_v, acc_ref, sem):
    wid = lax.axis_index("s") * _NC + lax.axis_index("c")
    cp = pltpu.async_copy(traj_hbm.at[pl.ds(wid * _TV, _TV)],
                          traj_v.at[pl.ds(0, _TV)], sem)
    pltpu.sync_copy(score_hbm.at[pl.ds(wid * (_RW * _M), _RW * _M)],
                    score_v.at[pl.ds(0, _RW * _M)])
    cp.wait()

    iota = lax.iota(jnp.int32, _L)
    iota2 = iota * 2
    pad_hi = iota >= (_H - _L)
    full = jnp.int32(2 * _L)
    zero = jnp.int32(0)

    def argmax_mark(s0, s1, s2, s3):
        m = jnp.maximum(jnp.maximum(s0, s1), jnp.maximum(s2, s3))
        mx = jnp.max(m)
        e0 = s0 == mx
        e1 = s1 == mx
        e2 = s2 == mx
        e3 = s3 == mx
        f0 = plsc.all_reduce_ffs(e0)
        f1 = plsc.all_reduce_ffs(e1)
        f2 = plsc.all_reduce_ffs(e2)
        f3 = plsc.all_reduce_ffs(e3)
        p0 = plsc.all_reduce_population_count(e0)
        p1 = plsc.all_reduce_population_count(e1)
        p2 = plsc.all_reduce_population_count(e2)
        c = jnp.where(p0 > zero, f0,
                      jnp.where(p1 > zero, f1 + 16,
                                jnp.where(p2 > zero, f2 + 32, f3 + 48)))
        s0 = jnp.where(iota == c, _NEG, s0)
        s1 = jnp.where(iota + 16 == c, _NEG, s1)
        s2 = jnp.where(iota + 32 == c, _NEG, s2)
        s3 = jnp.where(iota + 48 == c, _NEG, s3)
        return c[0], s0, s1, s2, s3

    def deinterleave(cb):
        xa0 = plsc.load_gather(traj_v, [cb + iota2])
        xa1 = plsc.load_gather(traj_v, [cb + 32 + iota2])
        ya0 = plsc.load_gather(traj_v, [cb + 1 + iota2])
        ya1 = plsc.load_gather(traj_v, [cb + 33 + iota2])
        return xa0, xa1, ya0, ya1

    def keep_store(parity, slot, xa0, xa1, ya0, ya1):
        ko = parity * ((_K + 1) * 64) + slot * 64
        kept_v[pl.ds(ko, _L)] = xa0
        kept_v[pl.ds(ko + 16, _L)] = xa1
        kept_v[pl.ds(ko + 32, _L)] = ya0
        kept_v[pl.ds(ko + 48, _L)] = ya1

    def row_init(r, parity):
        para = parity * (_K + 2)
        sb = r * _M
        s0 = score_v[pl.ds(sb, _L)]
        s1 = score_v[pl.ds(sb + _L, _L)]
        s2 = score_v[pl.ds(sb + 2 * _L, _L)]
        s3 = score_v[pl.ds(sb + 3 * _L, _L)]
        s3 = jnp.where(iota < (_M - 3 * _L), s3, _NEG)
        for j in range(_K + 2):
            acc_ref[para + j] = jnp.int32(0)
        c0, s0, s1, s2, s3 = argmax_mark(s0, s1, s2, s3)
        cb0 = r * (_M * _D) + c0 * _D
        acc_ref[para] = cb0
        keep_store(parity, 0, *deinterleave(cb0))
        return (jnp.int32(1), jnp.int32(1), s0, s1, s2, s3)

    def visit(st, r, parity):
        para = parity * (_K + 2)
        visited, cnt, s0, s1, s2, s3 = st
        active = jnp.logical_and(visited < _M, cnt < _K)
        c, n0, n1, n2, n3 = argmax_mark(s0, s1, s2, s3)
        s0 = jnp.where(active, n0, s0)
        s1 = jnp.where(active, n1, s1)
        s2 = jnp.where(active, n2, s2)
        s3 = jnp.where(active, n3, s3)
        cb = r * (_M * _D) + c * _D
        xa0, xa1, ya0, ya1 = deinterleave(cb)
        sup = jnp.full((_L,), False)
        for j in range(_K):
            ko = parity * ((_K + 1) * 64) + j * 64
            xb0 = kept_v[pl.ds(ko, _L)]
            xb1 = kept_v[pl.ds(ko + 16, _L)]
            yb0 = kept_v[pl.ds(ko + 32, _L)]
            yb1 = kept_v[pl.ds(ko + 48, _L)]
            dx0 = xa0 - xb0
            dy0 = ya0 - yb0
            dx1 = xa1 - xb1
            dy1 = ya1 - yb1
            d0 = dx0 * dx0 + dy0 * dy0
            d1 = dx1 * dx1 + dy1 * dy1
            near0 = d0 < _THR
            near1 = jnp.logical_or(d1 < _THR, pad_hi)
            pc = (plsc.all_reduce_population_count(near0)
                  + plsc.all_reduce_population_count(near1))
            close = pc == full
            sup = jnp.logical_or(sup, jnp.logical_and(close, j < cnt))
        take = jnp.logical_and(
            active, plsc.all_reduce_population_count(sup)[0] == zero)
        slot = jnp.where(take, cnt, jnp.int32(_K))
        keep_store(parity, slot, xa0, xa1, ya0, ya1)
        acc_ref[para + jnp.where(take, cnt, jnp.int32(_K + 1))] = cb
        cnt = cnt + take.astype(jnp.int32)
        return (visited + active.astype(jnp.int32), cnt, s0, s1, s2, s3)

    def write_out(r, parity, cntf):
        para = parity * (_K + 2)
        ob_row = r * (_K * _D)
        for j in range(_K):
            bj = acc_ref[para + j]
            valid = j < cntf
            for p in (0, 16, 32, 44):
                v = traj_v[pl.ds(bj + p, _L)]
                v = jnp.where(valid, v, 0.0)
                sel_v[pl.ds(ob_row + j * _D + p, _L)] = v

    def pair_body(p, cnts):
        rA = 2 * p
        rB = 2 * p + 1
        stA = row_init(rA, 0)
        stB = row_init(rB, 1)

        def cond(st):
            runA = jnp.logical_and(st[0] < _M, st[1] < _K)
            runB = jnp.logical_and(st[6] < _M, st[7] < _K)
            return jnp.logical_or(runA, runB)

        def body(st):
            nA = visit(st[0:6], rA, 0)
            nB = visit(st[6:12], rB, 1)
            return nA + nB

        st = lax.while_loop(cond, body, stA + stB)
        cntfA = st[1]
        cntfB = st[7]
        write_out(rA, 0, cntfA)
        write_out(rB, 1, cntfB)
        cnts = jnp.where(iota == rA, cntfA, cnts)
        return jnp.where(iota == rB, cntfB, cnts)

    cnts = lax.fori_loop(0, _RW // 2, pair_body, jnp.zeros((_L,), jnp.int32))
    cnt_v[...] = cnts
    pltpu.sync_copy(sel_v, sel_hbm.at[pl.ds(wid * _SEL, _SEL)])
    pltpu.sync_copy(cnt_v.at[pl.ds(0, _RW)], cnt_hbm.at[pl.ds(wid * _RW, _RW)])


def _sc_select(traj_flat, score_flat):
    mesh = plsc.VectorSubcoreMesh(core_axis_name="c", subcore_axis_name="s",
                                  num_cores=_NC, num_subcores=_NS)
    f = pl.kernel(
        _sc_body,
        out_type=(
            jax.ShapeDtypeStruct((_B * _K * _D,), jnp.float32),
            jax.ShapeDtypeStruct((_B,), jnp.int32),
        ),
        mesh=mesh,
        scratch_types=[
            pltpu.VMEM((_TV + 64,), jnp.float32),
            pltpu.VMEM((_RW * _M + 16,), jnp.float32),
            pltpu.VMEM((_SEL,), jnp.float32),
            pltpu.VMEM((_L,), jnp.int32),
            pltpu.VMEM((2 * (_K + 1) * 64,), jnp.float32),
            pltpu.SMEM((2 * (_K + 2),), jnp.int32),
            pltpu.SemaphoreType.DMA,
        ],
        compiler_params=pltpu.CompilerParams(needs_layout_passes=False),
    )
    return f(traj_flat, score_flat)


def _tc_mask_body(sel_ref, cnt_ref, out_ref):
    sel = sel_ref[...]
    cnt = cnt_ref[...]
    bi = lax.broadcasted_iota(jnp.int32, (_B, _B), 0)
    bj = lax.broadcasted_iota(jnp.int32, (_B, _B), 1)
    m = jnp.where(bj > bi, jnp.broadcast_to(cnt, (_B, _B)), jnp.int32(_K))
    suffix_after = jnp.min(m, axis=1)
    limit = jnp.minimum(cnt[0], suffix_after)
    col = lax.broadcasted_iota(jnp.int32, (_B, _K * _D), 1) // _D
    mask = col < limit[:, None]
    out_ref[...] = jnp.where(mask, sel, 0.0)


_tc_mask = pl.pallas_call(
    _tc_mask_body,
    out_shape=jax.ShapeDtypeStruct((_B, _K * _D), jnp.float32),
)


def kernel(traj_in, score):
    sel_flat, cnt = _sc_select(traj_in.reshape(-1), score.reshape(-1))
    out = _tc_mask(sel_flat.reshape(_B, _K * _D), cnt.reshape(1, _B))
    return out.reshape(_B, _K, _D)


# speedup vs baseline: 1.0042x; 1.0010x over previous
import jax
import jax.numpy as jnp
from jax import lax
from jax.experimental import pallas as pl
from jax.experimental.pallas import tpu as pltpu
from jax.experimental.pallas import tpu_sc as plsc

_B, _M, _D = 256, 50, 60
_H = _D // 2
_K = 6
_THR = 0.2
_L = 16
_NC, _NS = 2, 16
_NW = _NC * _NS
_RW = _B // _NW
_TV = _RW * _M * _D
_SEL = _RW * _K * _D
_NEG = float("-inf")


def _sc_body(traj_hbm, score_hbm, sel_hbm, cnt_hbm,
             traj_v, score_v, sel_v, cnt_v, kept_v, acc_ref, sem):
    wid = lax.axis_index("s") * _NC + lax.axis_index("c")
    cp = pltpu.async_copy(traj_hbm.at[pl.ds(wid * _TV, _TV)],
                          traj_v.at[pl.ds(0, _TV)], sem)
    pltpu.sync_copy(score_hbm.at[pl.ds(wid * (_RW * _M), _RW * _M)],
                    score_v.at[pl.ds(0, _RW * _M)])
    cp.wait()

    iota = lax.iota(jnp.int32, _L)
    iota2 = iota * 2
    pad_hi = iota >= (_H - _L)
    full = jnp.int32(2 * _L)
    zero = jnp.int32(0)

    def argmax_mark(s0, s1, s2, s3):
        m = jnp.maximum(jnp.maximum(s0, s1), jnp.maximum(s2, s3))
        mx = jnp.max(m)
        e0 = s0 == mx
        e1 = s1 == mx
        e2 = s2 == mx
        e3 = s3 == mx
        f0 = plsc.all_reduce_ffs(e0)
        f1 = plsc.all_reduce_ffs(e1)
        f2 = plsc.all_reduce_ffs(e2)
        f3 = plsc.all_reduce_ffs(e3)
        p0 = plsc.all_reduce_population_count(e0)
        p1 = plsc.all_reduce_population_count(e1)
        p2 = plsc.all_reduce_population_count(e2)
        c = jnp.where(p0 > zero, f0,
                      jnp.where(p1 > zero, f1 + 16,
                                jnp.where(p2 > zero, f2 + 32, f3 + 48)))
        s0 = jnp.where(iota == c, _NEG, s0)
        s1 = jnp.where(iota + 16 == c, _NEG, s1)
        s2 = jnp.where(iota + 32 == c, _NEG, s2)
        s3 = jnp.where(iota + 48 == c, _NEG, s3)
        return c[0], s0, s1, s2, s3

    def deinterleave(cb):
        xa0 = plsc.load_gather(traj_v, [cb + iota2])
        xa1 = plsc.load_gather(traj_v, [cb + 32 + iota2])
        ya0 = plsc.load_gather(traj_v, [cb + 1 + iota2])
        ya1 = plsc.load_gather(traj_v, [cb + 33 + iota2])
        return xa0, xa1, ya0, ya1

    def keep_store(slot, xa0, xa1, ya0, ya1):
        ko = slot * 64
        kept_v[pl.ds(ko, _L)] = xa0
        kept_v[pl.ds(ko + 16, _L)] = xa1
        kept_v[pl.ds(ko + 32, _L)] = ya0
        kept_v[pl.ds(ko + 48, _L)] = ya1

    def row_body(r, cnts):
        sb = r * _M
        s0 = score_v[pl.ds(sb, _L)]
        s1 = score_v[pl.ds(sb + _L, _L)]
        s2 = score_v[pl.ds(sb + 2 * _L, _L)]
        s3 = score_v[pl.ds(sb + 3 * _L, _L)]
        s3 = jnp.where(iota < (_M - 3 * _L), s3, _NEG)

        tb_row = r * (_M * _D)
        for j in range(_K + 2):
            acc_ref[j] = jnp.int32(0)
        c0, s0, s1, s2, s3 = argmax_mark(s0, s1, s2, s3)
        cb0 = tb_row + c0 * _D
        acc_ref[0] = cb0
        keep_store(0, *deinterleave(cb0))

        def cond(st):
            return jnp.logical_and(st[0] < _M, st[1] < _K)

        def body(st):
            visited, cnt, s0, s1, s2, s3 = st
            c, s0, s1, s2, s3 = argmax_mark(s0, s1, s2, s3)
            cb = tb_row + c * _D
            xa0, xa1, ya0, ya1 = deinterleave(cb)
            sup = jnp.full((_L,), False)
            for j in range(_K):
                ko = j * 64
                xb0 = kept_v[pl.ds(ko, _L)]
                xb1 = kept_v[pl.ds(ko + 16, _L)]
                yb0 = kept_v[pl.ds(ko + 32, _L)]
                yb1 = kept_v[pl.ds(ko + 48, _L)]
                dx0 = xa0 - xb0
                dy0 = ya0 - yb0
                dx1 = xa1 - xb1
                dy1 = ya1 - yb1
                d0 = dx0 * dx0 + dy0 * dy0
                d1 = dx1 * dx1 + dy1 * dy1
                near0 = d0 < _THR
                near1 = jnp.logical_or(d1 < _THR, pad_hi)
                pc = (plsc.all_reduce_population_count(near0)
                      + plsc.all_reduce_population_count(near1))
                close = pc == full
                sup = jnp.logical_or(sup, jnp.logical_and(close, j < cnt))
            take = plsc.all_reduce_population_count(sup)[0] == zero
            slot = jnp.where(take, cnt, jnp.int32(_K))
            keep_store(slot, xa0, xa1, ya0, ya1)
            acc_ref[jnp.where(take, cnt, jnp.int32(_K + 1))] = cb
            cnt = cnt + take.astype(jnp.int32)
            return (visited + 1, cnt, s0, s1, s2, s3)

        st = lax.while_loop(cond, body,
                            (jnp.int32(1), jnp.int32(1), s0, s1, s2, s3))
        cntf = st[1]

        ob_row = r * (_K * _D)
        for j in range(_K):
            bj = acc_ref[j]
            valid = j < cntf
            for p in (0, 16, 32, 44):
                v = traj_v[pl.ds(bj + p, _L)]
                v = jnp.where(valid, v, 0.0)
                sel_v[pl.ds(ob_row + j * _D + p, _L)] = v
        return jnp.where(iota == r, cntf, cnts)

    cnts = lax.fori_loop(0, _RW, row_body, jnp.zeros((_L,), jnp.int32))
    cnt_v[...] = cnts
    cpo = pltpu.async_copy(sel_v, sel_hbm.at[pl.ds(wid * _SEL, _SEL)], sem)
    pltpu.sync_copy(cnt_v.at[pl.ds(0, _RW)], cnt_hbm.at[pl.ds(wid * _RW, _RW)])
    cpo.wait()


def _sc_select(traj_flat, score_flat):
    mesh = plsc.VectorSubcoreMesh(core_axis_name="c", subcore_axis_name="s",
                                  num_cores=_NC, num_subcores=_NS)
    f = pl.kernel(
        _sc_body,
        out_type=(
            jax.ShapeDtypeStruct((_B * _K * _D,), jnp.float32),
            jax.ShapeDtypeStruct((_B,), jnp.int32),
        ),
        mesh=mesh,
        scratch_types=[
            pltpu.VMEM((_TV + 64,), jnp.float32),
            pltpu.VMEM((_RW * _M + 16,), jnp.float32),
            pltpu.VMEM((_SEL,), jnp.float32),
            pltpu.VMEM((_L,), jnp.int32),
            pltpu.VMEM(((_K + 1) * 64,), jnp.float32),
            pltpu.SMEM((_K + 2,), jnp.int32),
            pltpu.SemaphoreType.DMA,
        ],
        compiler_params=pltpu.CompilerParams(needs_layout_passes=False),
    )
    return f(traj_flat, score_flat)


def _tc_mask_body(sel_ref, cnt_ref, out_ref):
    sel = sel_ref[...]
    cnt = cnt_ref[...]
    bi = lax.broadcasted_iota(jnp.int32, (_B, _B), 0)
    bj = lax.broadcasted_iota(jnp.int32, (_B, _B), 1)
    m = jnp.where(bj > bi, jnp.broadcast_to(cnt, (_B, _B)), jnp.int32(_K))
    suffix_after = jnp.min(m, axis=1)
    limit = jnp.minimum(cnt[0], suffix_after)
    col = lax.broadcasted_iota(jnp.int32, (_B, _K * _D), 1) // _D
    mask = col < limit[:, None]
    out_ref[...] = jnp.where(mask, sel, 0.0)


_tc_mask = pl.pallas_call(
    _tc_mask_body,
    out_shape=jax.ShapeDtypeStruct((_B, _K * _D), jnp.float32),
)


def kernel(traj_in, score):
    sel_flat, cnt = _sc_select(traj_in.reshape(-1), score.reshape(-1))
    out = _tc_mask(sel_flat.reshape(_B, _K * _D), cnt.reshape(1, _B))
    return out.reshape(_B, _K, _D)


# speedup vs baseline: 1.0061x; 1.0019x over previous
import jax
import jax.numpy as jnp
from jax import lax
from jax.experimental import pallas as pl
from jax.experimental.pallas import tpu as pltpu
from jax.experimental.pallas import tpu_sc as plsc

_B, _M, _D = 256, 50, 60
_H = _D // 2
_K = 6
_THR = 0.2
_L = 16
_NC, _NS = 2, 16
_NW = _NC * _NS
_RW = _B // _NW
_TV = _RW * _M * _D
_SEL = _RW * _K * _D
_NEG = float("-inf")


_TV2 = 2 * _M * _D


def _sc_body(traj_hbm, score_hbm, sel_hbm, cnt_hbm,
             traj_v, score_v, sel_v, cnt_v, kept_v, acc_ref, sem, sem2):
    wid = lax.axis_index("s") * _NC + lax.axis_index("c")
    cp1 = pltpu.async_copy(traj_hbm.at[pl.ds(wid * _TV, _TV2)],
                           traj_v.at[pl.ds(0, _TV2)], sem)
    cp2 = pltpu.async_copy(traj_hbm.at[pl.ds(wid * _TV + _TV2, _TV - _TV2)],
                           traj_v.at[pl.ds(_TV2, _TV - _TV2)], sem2)
    pltpu.sync_copy(score_hbm.at[pl.ds(wid * (_RW * _M), _RW * _M)],
                    score_v.at[pl.ds(0, _RW * _M)])
    cp1.wait()

    iota = lax.iota(jnp.int32, _L)
    iota2 = iota * 2
    pad_hi = iota >= (_H - _L)
    full = jnp.int32(2 * _L)
    zero = jnp.int32(0)

    def argmax_mark(s0, s1, s2, s3):
        m = jnp.maximum(jnp.maximum(s0, s1), jnp.maximum(s2, s3))
        mx = jnp.max(m)
        e0 = s0 == mx
        e1 = s1 == mx
        e2 = s2 == mx
        e3 = s3 == mx
        f0 = plsc.all_reduce_ffs(e0)
        f1 = plsc.all_reduce_ffs(e1)
        f2 = plsc.all_reduce_ffs(e2)
        f3 = plsc.all_reduce_ffs(e3)
        p0 = plsc.all_reduce_population_count(e0)
        p1 = plsc.all_reduce_population_count(e1)
        p2 = plsc.all_reduce_population_count(e2)
        c = jnp.where(p0 > zero, f0,
                      jnp.where(p1 > zero, f1 + 16,
                                jnp.where(p2 > zero, f2 + 32, f3 + 48)))
        s0 = jnp.where(iota == c, _NEG, s0)
        s1 = jnp.where(iota + 16 == c, _NEG, s1)
        s2 = jnp.where(iota + 32 == c, _NEG, s2)
        s3 = jnp.where(iota + 48 == c, _NEG, s3)
        return c[0], s0, s1, s2, s3

    def deinterleave(cb):
        xa0 = plsc.load_gather(traj_v, [cb + iota2])
        xa1 = plsc.load_gather(traj_v, [cb + 32 + iota2])
        ya0 = plsc.load_gather(traj_v, [cb + 1 + iota2])
        ya1 = plsc.load_gather(traj_v, [cb + 33 + iota2])
        return xa0, xa1, ya0, ya1

    def keep_store(slot, xa0, xa1, ya0, ya1):
        ko = slot * 64
        kept_v[pl.ds(ko, _L)] = xa0
        kept_v[pl.ds(ko + 16, _L)] = xa1
        kept_v[pl.ds(ko + 32, _L)] = ya0
        kept_v[pl.ds(ko + 48, _L)] = ya1

    def row_body(r, cnts):
        sb = r * _M
        s0 = score_v[pl.ds(sb, _L)]
        s1 = score_v[pl.ds(sb + _L, _L)]
        s2 = score_v[pl.ds(sb + 2 * _L, _L)]
        s3 = score_v[pl.ds(sb + 3 * _L, _L)]
        s3 = jnp.where(iota < (_M - 3 * _L), s3, _NEG)

        tb_row = r * (_M * _D)
        for j in range(_K + 2):
            acc_ref[j] = jnp.int32(0)
        c0, s0, s1, s2, s3 = argmax_mark(s0, s1, s2, s3)
        cb0 = tb_row + c0 * _D
        acc_ref[0] = cb0
        keep_store(0, *deinterleave(cb0))

        def cond(st):
            return jnp.logical_and(st[0] < _M, st[1] < _K)

        def body(st):
            visited, cnt, s0, s1, s2, s3 = st
            c, s0, s1, s2, s3 = argmax_mark(s0, s1, s2, s3)
            cb = tb_row + c * _D
            xa0, xa1, ya0, ya1 = deinterleave(cb)
            sup = jnp.full((_L,), False)
            for j in range(_K):
                ko = j * 64
                xb0 = kept_v[pl.ds(ko, _L)]
                xb1 = kept_v[pl.ds(ko + 16, _L)]
                yb0 = kept_v[pl.ds(ko + 32, _L)]
                yb1 = kept_v[pl.ds(ko + 48, _L)]
                dx0 = xa0 - xb0
                dy0 = ya0 - yb0
                dx1 = xa1 - xb1
                dy1 = ya1 - yb1
                d0 = dx0 * dx0 + dy0 * dy0
                d1 = dx1 * dx1 + dy1 * dy1
                near0 = d0 < _THR
                near1 = jnp.logical_or(d1 < _THR, pad_hi)
                pc = (plsc.all_reduce_population_count(near0)
                      + plsc.all_reduce_population_count(near1))
                close = pc == full
                sup = jnp.logical_or(sup, jnp.logical_and(close, j < cnt))
            take = plsc.all_reduce_population_count(sup)[0] == zero
            slot = jnp.where(take, cnt, jnp.int32(_K))
            keep_store(slot, xa0, xa1, ya0, ya1)
            acc_ref[jnp.where(take, cnt, jnp.int32(_K + 1))] = cb
            cnt = cnt + take.astype(jnp.int32)
            return (visited + 1, cnt, s0, s1, s2, s3)

        st = lax.while_loop(cond, body,
                            (jnp.int32(1), jnp.int32(1), s0, s1, s2, s3))
        cntf = st[1]

        ob_row = r * (_K * _D)
        for j in range(_K):
            bj = acc_ref[j]
            valid = j < cntf
            for p in (0, 16, 32, 44):
                v = traj_v[pl.ds(bj + p, _L)]
                v = jnp.where(valid, v, 0.0)
                sel_v[pl.ds(ob_row + j * _D + p, _L)] = v
        return jnp.where(iota == r, cntf, cnts)

    cnts = lax.fori_loop(0, 2, row_body, jnp.zeros((_L,), jnp.int32))
    cp2.wait()
    cnts = lax.fori_loop(2, _RW, row_body, cnts)
    cnt_v[...] = cnts
    cpo = pltpu.async_copy(sel_v, sel_hbm.at[pl.ds(wid * _SEL, _SEL)], sem)
    pltpu.sync_copy(cnt_v.at[pl.ds(0, _RW)], cnt_hbm.at[pl.ds(wid * _RW, _RW)])
    cpo.wait()


def _sc_select(traj_flat, score_flat):
    mesh = plsc.VectorSubcoreMesh(core_axis_name="c", subcore_axis_name="s",
                                  num_cores=_NC, num_subcores=_NS)
    f = pl.kernel(
        _sc_body,
        out_type=(
            jax.ShapeDtypeStruct((_B * _K * _D,), jnp.float32),
            jax.ShapeDtypeStruct((_B,), jnp.int32),
        ),
        mesh=mesh,
        scratch_types=[
            pltpu.VMEM((_TV + 64,), jnp.float32),
            pltpu.VMEM((_RW * _M + 16,), jnp.float32),
            pltpu.VMEM((_SEL,), jnp.float32),
            pltpu.VMEM((_L,), jnp.int32),
            pltpu.VMEM(((_K + 1) * 64,), jnp.float32),
            pltpu.SMEM((_K + 2,), jnp.int32),
            pltpu.SemaphoreType.DMA,
            pltpu.SemaphoreType.DMA,
        ],
        compiler_params=pltpu.CompilerParams(needs_layout_passes=False),
    )
    return f(traj_flat, score_flat)


def _tc_mask_body(sel_ref, cnt_ref, out_ref):
    sel = sel_ref[...]
    cnt = cnt_ref[...]
    bi = lax.broadcasted_iota(jnp.int32, (_B, _B), 0)
    bj = lax.broadcasted_iota(jnp.int32, (_B, _B), 1)
    m = jnp.where(bj > bi, jnp.broadcast_to(cnt, (_B, _B)), jnp.int32(_K))
    suffix_after = jnp.min(m, axis=1)
    limit = jnp.minimum(cnt[0], suffix_after)
    col = lax.broadcasted_iota(jnp.int32, (_B, _K * _D), 1) // _D
    mask = col < limit[:, None]
    out_ref[...] = jnp.where(mask, sel, 0.0)


_tc_mask = pl.pallas_call(
    _tc_mask_body,
    out_shape=jax.ShapeDtypeStruct((_B, _K * _D), jnp.float32),
)


def kernel(traj_in, score):
    sel_flat, cnt = _sc_select(traj_in.reshape(-1), score.reshape(-1))
    out = _tc_mask(sel_flat.reshape(_B, _K * _D), cnt.reshape(1, _B))
    return out.reshape(_B, _K, _D)
